# baseline (speedup 1.0000x reference)
import jax
import jax.numpy as jnp
from jax import lax
from jax.experimental import pallas as pl
from jax.experimental.pallas import tpu as pltpu
from jax.experimental.pallas import tpu_sc as plsc

NB = 8
LP = 1.0


def _recip_posint(xi):
    x = xi.astype(jnp.float32)
    e = jnp.int32(0)
    t = xi
    for k in (16, 8, 4, 2, 1):
        big = t >= (1 << k)
        e = e + jnp.where(big, k, 0)
        t = jnp.where(big, lax.shift_right_logical(t, k), t)
    r0 = jnp.float32(1.0)
    for k in (1, 2, 4, 8, 16):
        c = jnp.float32(2.0 ** -k)
        r0 = r0 * jnp.where((e & k) != 0, c, jnp.float32(1.0))
    r0 = r0 * jnp.float32(0.5)
    m = x * r0
    r = (jnp.float32(48.0 / 17.0) - jnp.float32(32.0 / 17.0) * m) * r0
    for _ in range(4):
        r = r * (jnp.float32(2.0) - x * r)
    return r


def _build(n_rows, row_len):
    mesh = plsc.ScalarSubcoreMesh(axis_name="c", num_cores=2)

    def body(sb_hbm, slp_hbm, ws_hbm, cl_hbm, len_hbm, hyp_hbm, items_hbm,
             sb_out_hbm, items_out_hbm, worst_out_hbm, len_out_hbm,
             s_sb, s_slp, s_ws, s_cl, s_lenin, s_sbo, s_wso, s_leno, sem):
        c1 = pltpu.async_copy(sb_hbm, s_sb, sem)
        c2 = pltpu.async_copy(slp_hbm, s_slp, sem)
        c3 = pltpu.async_copy(ws_hbm, s_ws, sem)
        c4 = pltpu.async_copy(cl_hbm, s_cl, sem)
        c5 = pltpu.async_copy(len_hbm, s_lenin, sem)
        c1.wait()
        c2.wait()
        c3.wait()
        c4.wait()
        c5.wait()

        slp = s_slp[0]
        worst = s_ws[0]
        len_state = s_lenin[0]

        score = slp * _recip_posint(s_cl[0])
        cond = jnp.logical_or(len_state < NB, score > worst)
        new_len = len_state + jnp.where(cond, 1, 0).astype(jnp.int32)
        idx = jnp.where(new_len > NB + 1, 0, NB)

        keys = [jnp.where(jnp.int32(j) == idx, score, s_sb[j])
                for j in range(n_rows)]
        vals = [jnp.int32(j) for j in range(n_rows)]
        net = [(0, 3), (1, 7), (2, 5), (4, 8),
               (0, 7), (2, 4), (3, 8), (5, 6),
               (0, 2), (1, 3), (4, 5), (7, 8),
               (1, 4), (3, 6), (5, 7),
               (0, 1), (2, 4), (3, 5), (6, 8),
               (2, 3), (4, 5), (6, 7),
               (1, 2), (3, 4), (5, 6)]
        for a, b in net:
            ka, kb, va, vb = keys[a], keys[b], vals[a], vals[b]
            swap = jnp.logical_or(
                kb < ka, jnp.logical_and(kb == ka, vb < va))
            keys[a] = jnp.where(swap, kb, ka)
            keys[b] = jnp.where(swap, ka, kb)
            vals[a] = jnp.where(swap, vb, va)
            vals[b] = jnp.where(swap, va, vb)

        worst_new = jnp.where(new_len > NB, keys[1],
                              jnp.minimum(worst, score))
        for j in range(n_rows):
            s_sbo[j] = jnp.where(cond, keys[j], s_sb[j])
        s_wso[0] = jnp.where(cond, worst_new, worst)
        s_leno[0] = new_len

        cid = lax.axis_index("c")
        half = n_rows // 2

        @pl.when(cid == 0)
        def _scalar_outs():
            pltpu.async_copy(s_sbo, sb_out_hbm, sem)
            pltpu.async_copy(s_wso, worst_out_hbm, sem)
            pltpu.async_copy(s_leno, len_out_hbm, sem)

        for j in range(n_rows):
            srcj = jnp.where(cond, vals[j], jnp.int32(j))
            use_hyp = jnp.logical_and(cond, vals[j] == idx)
            mine = cid == (0 if j < half else 1)

            @pl.when(jnp.logical_and(mine, use_hyp))
            def _copy_hyp(j=j):
                pltpu.async_copy(
                    hyp_hbm, items_out_hbm.at[pl.ds(j, 1)], sem)

            @pl.when(jnp.logical_and(mine, jnp.logical_not(use_hyp)))
            def _copy_row(j=j, srcj=srcj):
                pltpu.async_copy(
                    items_hbm.at[pl.ds(srcj, 1)],
                    items_out_hbm.at[pl.ds(j, 1)], sem)
        @pl.when(cid == 0)
        def _drain0():
            for j in range(half):
                pltpu.make_async_copy(
                    items_hbm.at[pl.ds(0, 1)],
                    items_out_hbm.at[pl.ds(j, 1)], sem).wait()
            pltpu.make_async_copy(sb_hbm, sb_out_hbm, sem).wait()
            pltpu.make_async_copy(ws_hbm, worst_out_hbm, sem).wait()
            pltpu.make_async_copy(len_hbm, len_out_hbm, sem).wait()

        @pl.when(cid == 1)
        def _drain1():
            for j in range(half, n_rows):
                pltpu.make_async_copy(
                    items_hbm.at[pl.ds(0, 1)],
                    items_out_hbm.at[pl.ds(j, 1)], sem).wait()

    return pl.kernel(
        body,
        mesh=mesh,
        compiler_params=pltpu.CompilerParams(needs_layout_passes=False),
        out_type=(
            jax.ShapeDtypeStruct((n_rows,), jnp.float32),
            jax.ShapeDtypeStruct((n_rows, row_len), jnp.int32),
            jax.ShapeDtypeStruct((1,), jnp.float32),
            jax.ShapeDtypeStruct((1,), jnp.int32),
        ),
        scratch_types=[
            pltpu.SMEM((9,), jnp.float32),
            pltpu.SMEM((1,), jnp.float32),
            pltpu.SMEM((1,), jnp.float32),
            pltpu.SMEM((1,), jnp.int32),
            pltpu.SMEM((1,), jnp.int32),
            pltpu.SMEM((9,), jnp.float32),
            pltpu.SMEM((1,), jnp.float32),
            pltpu.SMEM((1,), jnp.int32),
            pltpu.SemaphoreType.DMA,
        ],
    )


def kernel(hyp, sum_logprobs, cur_len, len_state, worst_score, scoreboard,
           scoreboard_items):
    n_rows, row_len = scoreboard_items.shape
    cl1 = jnp.asarray(cur_len, jnp.int32).reshape(1)
    len1 = jnp.asarray(len_state, jnp.int32).reshape(1)
    hyp2 = hyp.reshape(1, row_len)

    sb_out, items_out, worst_out, len_out = _build(n_rows, row_len)(
        scoreboard, sum_logprobs, worst_score, cl1, len1, hyp2,
        scoreboard_items)
    return (sb_out, items_out, worst_out, len_out.reshape(()))


# speedup vs baseline: 1.0423x; 1.0423x over previous
import jax
import jax.numpy as jnp
from jax import lax
from jax.experimental import pallas as pl
from jax.experimental.pallas import tpu as pltpu
from jax.experimental.pallas import tpu_sc as plsc

NB = 8
LP = 1.0


def _recip_posint(xi):
    x = xi.astype(jnp.float32)
    e = jnp.int32(0)
    t = xi
    for k in (16, 8, 4, 2, 1):
        big = t >= (1 << k)
        e = e + jnp.where(big, k, 0)
        t = jnp.where(big, lax.shift_right_logical(t, k), t)
    r0 = jnp.float32(1.0)
    for k in (1, 2, 4, 8, 16):
        c = jnp.float32(2.0 ** -k)
        r0 = r0 * jnp.where((e & k) != 0, c, jnp.float32(1.0))
    r0 = r0 * jnp.float32(0.5)
    m = x * r0
    r = (jnp.float32(48.0 / 17.0) - jnp.float32(32.0 / 17.0) * m) * r0
    for _ in range(4):
        r = r * (jnp.float32(2.0) - x * r)
    return r


def _build(n_rows, row_len):
    mesh = plsc.ScalarSubcoreMesh(axis_name="c", num_cores=1)

    def body(sb_hbm, slp_hbm, ws_hbm, cl_hbm, len_hbm, hyp_hbm, items_hbm,
             sb_out_hbm, items_out_hbm, worst_out_hbm, len_out_hbm,
             s_sb, s_slp, s_ws, s_cl, s_lenin, s_sbo, s_wso, s_leno, sem):
        c1 = pltpu.async_copy(sb_hbm, s_sb, sem)
        c2 = pltpu.async_copy(slp_hbm, s_slp, sem)
        c3 = pltpu.async_copy(ws_hbm, s_ws, sem)
        c4 = pltpu.async_copy(cl_hbm, s_cl, sem)
        c5 = pltpu.async_copy(len_hbm, s_lenin, sem)
        c4.wait()
        recip = _recip_posint(s_cl[0])
        c2.wait()
        c3.wait()
        c5.wait()
        slp = s_slp[0]
        worst = s_ws[0]
        len_state = s_lenin[0]

        score = slp * recip
        cond = jnp.logical_or(len_state < NB, score > worst)
        new_len = len_state + jnp.where(cond, 1, 0).astype(jnp.int32)
        idx = jnp.where(new_len > NB + 1, 0, NB)
        c1.wait()

        keys = [jnp.where(jnp.int32(j) == idx, score, s_sb[j])
                for j in range(n_rows)]
        vals = [jnp.int32(j) for j in range(n_rows)]
        net = [(0, 3), (1, 7), (2, 5), (4, 8),
               (0, 7), (2, 4), (3, 8), (5, 6),
               (0, 2), (1, 3), (4, 5), (7, 8),
               (1, 4), (3, 6), (5, 7),
               (0, 1), (2, 4), (3, 5), (6, 8),
               (2, 3), (4, 5), (6, 7),
               (1, 2), (3, 4), (5, 6)]
        for a, b in net:
            ka, kb, va, vb = keys[a], keys[b], vals[a], vals[b]
            swap = jnp.logical_or(
                kb < ka, jnp.logical_and(kb == ka, vb < va))
            keys[a] = jnp.where(swap, kb, ka)
            keys[b] = jnp.where(swap, ka, kb)
            vals[a] = jnp.where(swap, vb, va)
            vals[b] = jnp.where(swap, va, vb)

        worst_new = jnp.where(new_len > NB, keys[1],
                              jnp.minimum(worst, score))
        for j in range(n_rows):
            s_sbo[j] = jnp.where(cond, keys[j], s_sb[j])
        s_wso[0] = jnp.where(cond, worst_new, worst)
        s_leno[0] = new_len

        pending = [
            pltpu.async_copy(s_sbo, sb_out_hbm, sem),
            pltpu.async_copy(s_wso, worst_out_hbm, sem),
            pltpu.async_copy(s_leno, len_out_hbm, sem),
        ]
        for j in range(n_rows):
            srcj = jnp.where(cond, vals[j], jnp.int32(j))
            use_hyp = jnp.logical_and(cond, vals[j] == idx)

            @pl.when(use_hyp)
            def _copy_hyp(j=j):
                pltpu.async_copy(
                    hyp_hbm, items_out_hbm.at[pl.ds(j, 1)], sem)

            @pl.when(jnp.logical_not(use_hyp))
            def _copy_row(j=j, srcj=srcj):
                pltpu.async_copy(
                    items_hbm.at[pl.ds(srcj, 1)],
                    items_out_hbm.at[pl.ds(j, 1)], sem)
        for j in range(n_rows):
            pltpu.make_async_copy(
                items_hbm.at[pl.ds(0, 1)],
                items_out_hbm.at[pl.ds(j, 1)], sem).wait()
        for h in pending:
            h.wait()

    return pl.kernel(
        body,
        mesh=mesh,
        compiler_params=pltpu.CompilerParams(needs_layout_passes=False),
        out_type=(
            jax.ShapeDtypeStruct((n_rows,), jnp.float32),
            jax.ShapeDtypeStruct((n_rows, row_len), jnp.int32),
            jax.ShapeDtypeStruct((1,), jnp.float32),
            jax.ShapeDtypeStruct((1,), jnp.int32),
        ),
        scratch_types=[
            pltpu.SMEM((9,), jnp.float32),
            pltpu.SMEM((1,), jnp.float32),
            pltpu.SMEM((1,), jnp.float32),
            pltpu.SMEM((1,), jnp.int32),
            pltpu.SMEM((1,), jnp.int32),
            pltpu.SMEM((9,), jnp.float32),
            pltpu.SMEM((1,), jnp.float32),
            pltpu.SMEM((1,), jnp.int32),
            pltpu.SemaphoreType.DMA,
        ],
    )


def kernel(hyp, sum_logprobs, cur_len, len_state, worst_score, scoreboard,
           scoreboard_items):
    n_rows, row_len = scoreboard_items.shape
    cl1 = jnp.asarray(cur_len, jnp.int32).reshape(1)
    len1 = jnp.asarray(len_state, jnp.int32).reshape(1)
    hyp2 = hyp.reshape(1, row_len)

    sb_out, items_out, worst_out, len_out = _build(n_rows, row_len)(
        scoreboard, sum_logprobs, worst_score, cl1, len1, hyp2,
        scoreboard_items)
    return (sb_out, items_out, worst_out, len_out.reshape(()))
